# baseline (speedup 1.0000x reference)
import functools

import jax
import jax.numpy as jnp
from jax import lax
from jax.experimental import pallas as pl
from jax.experimental.pallas import tpu as pltpu
from jax.experimental.pallas import tpu_sc as plsc

N_NODES = 10000
N_EDGES = 320000
D_FEAT = 128

NUM_CORES = 2
NUM_SUBCORES = 16
LANES = 16
NUM_WORKERS = NUM_CORES * NUM_SUBCORES

E_PER_W = N_EDGES // NUM_WORKERS
CHUNK = 80
N_CHUNKS = E_PER_W // CHUNK
GROUPS = CHUNK // LANES


NBUF = 3


def _sc_body(h_hbm, u_hbm, v_hbm, out_hbm,
             uidx, vidx, scores, psum,
             urows0, urows1, urows2,
             vrows0, vrows1, vrows2,
             sem0, sem1, sem2):
    wid = lax.axis_index("s") * NUM_CORES + lax.axis_index("c")
    pltpu.sync_copy(u_hbm.at[wid], uidx)
    pltpu.sync_copy(v_hbm.at[wid], vidx)

    ubufs = (urows0, urows1, urows2)
    vbufs = (vrows0, vrows1, vrows2)
    sems = (sem0, sem1, sem2)

    def issue(i, b):
        pltpu.async_copy(h_hbm.at[uidx.at[i]], ubufs[b], sems[b])
        pltpu.async_copy(h_hbm.at[vidx.at[i]], vbufs[b], sems[b])

    def wait(b):
        pltpu.make_async_copy(h_hbm.at[pl.ds(0, CHUNK)], ubufs[b], sems[b]).wait()
        pltpu.make_async_copy(h_hbm.at[pl.ds(0, CHUNK)], vbufs[b], sems[b]).wait()

    lane_iota = lax.broadcasted_iota(jnp.int32, (LANES,), 0)

    def compute(b, i):
        ur, vr = ubufs[b], vbufs[b]

        def group_body(g, carry):
            tot = jnp.zeros((LANES,), jnp.float32)
            for el in range(LANES):
                e = g * LANES + el
                acc = ur[e, pl.ds(0, LANES)] * vr[e, pl.ds(0, LANES)]
                for k in range(1, D_FEAT // LANES):
                    acc = acc + (ur[e, pl.ds(k * LANES, LANES)]
                                 * vr[e, pl.ds(k * LANES, LANES)])
                tot = jnp.where(lane_iota == el,
                                jnp.broadcast_to(jnp.sum(acc), (LANES,)),
                                tot)
            scores[i, pl.ds(g * LANES, LANES)] = tot
            return carry

        lax.fori_loop(0, GROUPS, group_body, 0)

    for b in range(NBUF - 1):
        issue(b, b)

    def loop_body(j, carry):
        i0 = NBUF * j
        for t in range(NBUF):
            i = i0 + t
            wait(t)
            compute(t, i)
            nxt = i + NBUF - 1

            @pl.when(nxt < N_CHUNKS)
            def _():
                issue(nxt, (t + NBUF - 1) % NBUF)

        return carry

    lax.fori_loop(0, (N_CHUNKS - 1) // NBUF, loop_body, 0)
    tail = ((N_CHUNKS - 1) // NBUF) * NBUF
    for i in range(tail, N_CHUNKS):
        b = i % NBUF
        wait(b)
        compute(b, i)

    pltpu.sync_copy(scores, out_hbm.at[wid])


@jax.jit
def kernel(h, edge_index):
    ei = edge_index.astype(jnp.int32)
    u3 = ei[0].reshape(NUM_WORKERS, N_CHUNKS, CHUNK)
    v3 = ei[1].reshape(NUM_WORKERS, N_CHUNKS, CHUNK)

    mesh = plsc.VectorSubcoreMesh(
        core_axis_name="c", subcore_axis_name="s",
        num_cores=NUM_CORES, num_subcores=NUM_SUBCORES,
    )
    run = functools.partial(
        pl.kernel,
        out_type=jax.ShapeDtypeStruct((NUM_WORKERS, N_CHUNKS, CHUNK),
                                      jnp.float32),
        mesh=mesh,
        compiler_params=pltpu.CompilerParams(needs_layout_passes=False),
        scratch_types=[
            pltpu.VMEM((N_CHUNKS, CHUNK), jnp.int32),
            pltpu.VMEM((N_CHUNKS, CHUNK), jnp.int32),
            pltpu.VMEM((N_CHUNKS, CHUNK), jnp.float32),
            pltpu.VMEM((LANES, 17), jnp.float32),
        ] + [pltpu.VMEM((CHUNK, D_FEAT), jnp.float32)] * 6
          + [pltpu.SemaphoreType.DMA] * 3,
    )(_sc_body)
    out3 = run(h, u3, v3)
    return out3.reshape(N_EDGES)


# speedup vs baseline: 1.2463x; 1.2463x over previous
import functools

import jax
import jax.numpy as jnp
from jax import lax
from jax.experimental import pallas as pl
from jax.experimental.pallas import tpu as pltpu
from jax.experimental.pallas import tpu_sc as plsc

N_NODES = 10000
N_EDGES = 320000
D_FEAT = 128

NUM_CORES = 2
NUM_SUBCORES = 16
LANES = 16
NUM_WORKERS = NUM_CORES * NUM_SUBCORES

E_PER_W = N_EDGES // NUM_WORKERS
CHUNK = 80
N_CHUNKS = E_PER_W // CHUNK
GROUPS = CHUNK // LANES


NBUF = 3


def _sc_body(h_hbm, u_hbm, v_hbm, out_hbm,
             uidx, vidx, scores, psum,
             urows0, urows1, urows2,
             vrows0, vrows1, vrows2,
             sem0, sem1, sem2):
    wid = lax.axis_index("s") * NUM_CORES + lax.axis_index("c")
    pltpu.sync_copy(u_hbm.at[wid], uidx)
    pltpu.sync_copy(v_hbm.at[wid], vidx)

    ubufs = (urows0, urows1, urows2)
    vbufs = (vrows0, vrows1, vrows2)
    sems = (sem0, sem1, sem2)

    def issue(i, b):
        pltpu.async_copy(h_hbm.at[uidx.at[i]], ubufs[b], sems[b])
        pltpu.async_copy(h_hbm.at[vidx.at[i]], vbufs[b], sems[b])

    def wait(b):
        pltpu.make_async_copy(h_hbm.at[pl.ds(0, CHUNK)], ubufs[b], sems[b]).wait()
        pltpu.make_async_copy(h_hbm.at[pl.ds(0, CHUNK)], vbufs[b], sems[b]).wait()

    lane_iota = lax.broadcasted_iota(jnp.int32, (LANES,), 0)

    def compute(b, i):
        ur, vr = ubufs[b], vbufs[b]

        zeros = jnp.zeros((LANES,), jnp.float32)
        for g in range(GROUPS):
            scores[i, pl.ds(g * LANES, LANES)] = zeros

        def group_body(g, carry):
            for el in range(LANES):
                e = g * LANES + el
                acc = ur[e, pl.ds(0, LANES)] * vr[e, pl.ds(0, LANES)]
                for k in range(1, D_FEAT // LANES):
                    acc = acc + (ur[e, pl.ds(k * LANES, LANES)]
                                 * vr[e, pl.ds(k * LANES, LANES)])
                plsc.addupdate_scatter(
                    scores,
                    [jnp.full((LANES,), i, jnp.int32),
                     jnp.full((LANES,), e, jnp.int32)],
                    acc)
            return carry

        lax.fori_loop(0, GROUPS, group_body, 0)

    for b in range(NBUF - 1):
        issue(b, b)

    def loop_body(j, carry):
        i0 = NBUF * j
        for t in range(NBUF):
            i = i0 + t
            wait(t)
            compute(t, i)
            nxt = i + NBUF - 1

            @pl.when(nxt < N_CHUNKS)
            def _():
                issue(nxt, (t + NBUF - 1) % NBUF)

        return carry

    lax.fori_loop(0, (N_CHUNKS - 1) // NBUF, loop_body, 0)
    tail = ((N_CHUNKS - 1) // NBUF) * NBUF
    for i in range(tail, N_CHUNKS):
        b = i % NBUF
        wait(b)
        compute(b, i)

    pltpu.sync_copy(scores, out_hbm.at[wid])


@jax.jit
def kernel(h, edge_index):
    ei = edge_index.astype(jnp.int32)
    u3 = ei[0].reshape(NUM_WORKERS, N_CHUNKS, CHUNK)
    v3 = ei[1].reshape(NUM_WORKERS, N_CHUNKS, CHUNK)

    mesh = plsc.VectorSubcoreMesh(
        core_axis_name="c", subcore_axis_name="s",
        num_cores=NUM_CORES, num_subcores=NUM_SUBCORES,
    )
    run = functools.partial(
        pl.kernel,
        out_type=jax.ShapeDtypeStruct((NUM_WORKERS, N_CHUNKS, CHUNK),
                                      jnp.float32),
        mesh=mesh,
        compiler_params=pltpu.CompilerParams(needs_layout_passes=False),
        scratch_types=[
            pltpu.VMEM((N_CHUNKS, CHUNK), jnp.int32),
            pltpu.VMEM((N_CHUNKS, CHUNK), jnp.int32),
            pltpu.VMEM((N_CHUNKS, CHUNK), jnp.float32),
            pltpu.VMEM((LANES, 17), jnp.float32),
        ] + [pltpu.VMEM((CHUNK, D_FEAT), jnp.float32)] * 6
          + [pltpu.SemaphoreType.DMA] * 3,
    )(_sc_body)
    out3 = run(h, u3, v3)
    return out3.reshape(N_EDGES)


# speedup vs baseline: 1.7793x; 1.4277x over previous
import functools

import jax
import jax.numpy as jnp
from jax import lax
from jax.experimental import pallas as pl
from jax.experimental.pallas import tpu as pltpu
from jax.experimental.pallas import tpu_sc as plsc

N_NODES = 10000
N_EDGES = 320000
D_FEAT = 128

NUM_CORES = 2
NUM_SUBCORES = 16
LANES = 16
NUM_WORKERS = NUM_CORES * NUM_SUBCORES

E_PER_W = N_EDGES // NUM_WORKERS
CHUNK = 80
N_CHUNKS = E_PER_W // CHUNK
GROUPS = CHUNK // LANES


NBUF = 3


def _sc_body(h_hbm, u_hbm, v_hbm, out_hbm,
             uidx, vidx, scores, psum,
             urows0, urows1, urows2,
             vrows0, vrows1, vrows2,
             sem0, sem1, sem2):
    wid = lax.axis_index("s") * NUM_CORES + lax.axis_index("c")
    pltpu.sync_copy(u_hbm.at[wid], uidx)
    pltpu.sync_copy(v_hbm.at[wid], vidx)

    ubufs = (urows0, urows1, urows2)
    vbufs = (vrows0, vrows1, vrows2)
    sems = (sem0, sem1, sem2)

    def issue(i, b):
        pltpu.async_copy(h_hbm.at[uidx.at[i]], ubufs[b], sems[b])
        pltpu.async_copy(h_hbm.at[vidx.at[i]], vbufs[b], sems[b])

    def wait(b):
        pltpu.make_async_copy(h_hbm.at[pl.ds(0, CHUNK)], ubufs[b], sems[b]).wait()
        pltpu.make_async_copy(h_hbm.at[pl.ds(0, CHUNK)], vbufs[b], sems[b]).wait()

    lane_iota = lax.broadcasted_iota(jnp.int32, (LANES,), 0)

    _dnums = lax.GatherDimensionNumbers(
        offset_dims=(), collapsed_slice_dims=(0,), start_index_map=(0,))

    def take16(x, idx):
        return lax.gather(x, idx[:, None], _dnums, (1,),
                          mode=lax.GatherScatterMode.PROMISE_IN_BOUNDS)

    def compute(b, i):
        ur, vr = ubufs[b], vbufs[b]

        def group_body(g, carry):
            vecs = []
            for el in range(LANES):
                e = g * LANES + el
                acc = ur[e, pl.ds(0, LANES)] * vr[e, pl.ds(0, LANES)]
                for k in range(1, D_FEAT // LANES):
                    acc = acc + (ur[e, pl.ds(k * LANES, LANES)]
                                 * vr[e, pl.ds(k * LANES, LANES)])
                vecs.append(acc)
            for sh in (1, 2, 4, 8):
                mask = (lane_iota & sh) == 0
                perm = lane_iota ^ sh
                vecs = [
                    jnp.where(mask, a, take16(b, perm))
                    + jnp.where(mask, take16(a, perm), b)
                    for a, b in zip(vecs[0::2], vecs[1::2])
                ]
            scores[i, pl.ds(g * LANES, LANES)] = vecs[0]
            return carry

        lax.fori_loop(0, GROUPS, group_body, 0)

    for b in range(NBUF - 1):
        issue(b, b)

    def loop_body(j, carry):
        i0 = NBUF * j
        for t in range(NBUF):
            i = i0 + t
            wait(t)
            compute(t, i)
            nxt = i + NBUF - 1

            @pl.when(nxt < N_CHUNKS)
            def _():
                issue(nxt, (t + NBUF - 1) % NBUF)

        return carry

    lax.fori_loop(0, (N_CHUNKS - 1) // NBUF, loop_body, 0)
    tail = ((N_CHUNKS - 1) // NBUF) * NBUF
    for i in range(tail, N_CHUNKS):
        b = i % NBUF
        wait(b)
        compute(b, i)

    pltpu.sync_copy(scores, out_hbm.at[wid])


@jax.jit
def kernel(h, edge_index):
    ei = edge_index.astype(jnp.int32)
    u3 = ei[0].reshape(NUM_WORKERS, N_CHUNKS, CHUNK)
    v3 = ei[1].reshape(NUM_WORKERS, N_CHUNKS, CHUNK)

    mesh = plsc.VectorSubcoreMesh(
        core_axis_name="c", subcore_axis_name="s",
        num_cores=NUM_CORES, num_subcores=NUM_SUBCORES,
    )
    run = functools.partial(
        pl.kernel,
        out_type=jax.ShapeDtypeStruct((NUM_WORKERS, N_CHUNKS, CHUNK),
                                      jnp.float32),
        mesh=mesh,
        compiler_params=pltpu.CompilerParams(needs_layout_passes=False),
        scratch_types=[
            pltpu.VMEM((N_CHUNKS, CHUNK), jnp.int32),
            pltpu.VMEM((N_CHUNKS, CHUNK), jnp.int32),
            pltpu.VMEM((N_CHUNKS, CHUNK), jnp.float32),
            pltpu.VMEM((LANES, 17), jnp.float32),
        ] + [pltpu.VMEM((CHUNK, D_FEAT), jnp.float32)] * 6
          + [pltpu.SemaphoreType.DMA] * 3,
    )(_sc_body)
    out3 = run(h, u3, v3)
    return out3.reshape(N_EDGES)
